# baseline (speedup 1.0000x reference)
import functools

import jax
import jax.numpy as jnp
from jax import lax
from jax.experimental import pallas as pl
from jax.experimental.pallas import tpu as pltpu
from jax.experimental.pallas import tpu_sc as plsc

N = 10000
N_PAD = 10240
E = 320000
NC = 2
NS = 16
K = 80
CHT = E // (NS * K)
CHW = CHT // NC
NPT = N_PAD // NS
RB = 5

_mesh = lambda: plsc.VectorSubcoreMesh(core_axis_name="c", subcore_axis_name="s")
_sc_params = lambda: pltpu.CompilerParams(use_tc_tiling_on_sc=False)


@functools.lru_cache(maxsize=None)
def _make_deg():
    @functools.partial(
        pl.kernel,
        out_type=jax.ShapeDtypeStruct((N_PAD, 128), jnp.float32),
        mesh=_mesh(),
        compiler_params=_sc_params(),
        scratch_types=[
            pltpu.VMEM((CHW, K), jnp.int32),
            pltpu.VMEM((K, 8), jnp.float32),
            pltpu.VMEM_SHARED((N_PAD, 8), jnp.float32),
            pltpu.SemaphoreType.DMA,
        ],
    )
    def deg_kernel(dst_hbm, ones_hbm, zeros_hbm, out_hbm, dstv, ones_v, acc, ssem):
        c = lax.axis_index("c")
        s = lax.axis_index("s")

        pltpu.sync_copy(zeros_hbm.at[pl.ds(s * NPT, NPT)], acc.at[pl.ds(s * NPT, NPT)])
        pltpu.sync_copy(ones_hbm, ones_v)
        pltpu.sync_copy(dst_hbm.at[s, pl.ds(c * CHW, CHW)], dstv)
        plsc.subcore_barrier()

        def group(g, _):
            for t in range(RB):
                pltpu.async_copy(ones_v, acc.at[dstv.at[g * RB + t]], ssem,
                                 add=True)
            for t in range(RB):
                pltpu.make_async_copy(ones_v, acc.at[dstv.at[0]], ssem).wait()
            return 0

        lax.fori_loop(0, CHW // RB, group, 0)
        plsc.subcore_barrier()
        pltpu.sync_copy(acc.at[pl.ds(s * NPT, NPT)],
                        out_hbm.at[pl.ds(s * NPT, NPT), pl.ds(c * 8, 8)])

    return deg_kernel


def _edge_pipeline(mk_src, srcv, dstv, rows, gsems, ssems, acc, n_chunks):
    M = n_chunks // RB

    def gissue(j, b):
        pltpu.async_copy(mk_src(j), rows[b], gsems[b])

    def gwait(b):
        pltpu.make_async_copy(mk_src(0), rows[b], gsems[b]).wait()

    def sissue(j, b):
        pltpu.async_copy(rows[b], acc.at[dstv.at[j]], ssems[b], add=True)

    def swait(b):
        pltpu.make_async_copy(rows[b], acc.at[dstv.at[0]], ssems[b]).wait()

    for b in range(3):
        gissue(b, b)
    for b in range(RB):
        b2 = (b - 2) % RB
        if b >= 2:
            swait(b2)
        gissue(b + 3, b2)
        gwait(b)
        sissue(b, b)

    def step(m, _):
        for b in range(RB):
            j = m * RB + b
            b2 = (b - 2) % RB
            swait(b2)
            gissue(j + 3, b2)
            gwait(b)
            sissue(j, b)
        return 0

    lax.fori_loop(1, M - 1, step, 0)

    base = (M - 1) * RB
    for b in range(RB):
        if b < 2:
            b2 = (b - 2) % RB
            swait(b2)
            gissue(base + b + 3, b2)
        gwait(b)
        sissue(base + b, b)
    for b in range(RB):
        swait(b)


def _prop_scratch(n_chunks):
    return [
        pltpu.VMEM((n_chunks, K), jnp.int32),
        pltpu.VMEM((n_chunks, K), jnp.int32),
    ] + [pltpu.VMEM((K, 64), jnp.float32) for _ in range(RB)] + [
        pltpu.VMEM_SHARED((N_PAD, 64), jnp.float32),
    ] + [pltpu.SemaphoreType.DMA for _ in range(2 * RB)]


@functools.lru_cache(maxsize=None)
def _make_prop_col():
    @functools.partial(
        pl.kernel,
        out_type=jax.ShapeDtypeStruct((N_PAD, 128), jnp.float32),
        mesh=_mesh(),
        compiler_params=_sc_params(),
        scratch_types=_prop_scratch(CHT),
    )
    def prop_kernel(tabA_hbm, tabB_hbm, src_hbm, dst_hbm, zeros_hbm, out_hbm,
                    srcv, dstv, r0, r1, r2, r3, r4, acc,
                    g0, g1, g2, g3, g4, s0, s1, s2, s3, s4):
        c = lax.axis_index("c")
        s = lax.axis_index("s")
        rows = [r0, r1, r2, r3, r4]
        gsems = [g0, g1, g2, g3, g4]
        ssems = [s0, s1, s2, s3, s4]

        pltpu.sync_copy(zeros_hbm.at[pl.ds(s * NPT, NPT)], acc.at[pl.ds(s * NPT, NPT)])
        pltpu.sync_copy(src_hbm.at[s], srcv)
        pltpu.sync_copy(dst_hbm.at[s], dstv)
        plsc.subcore_barrier()

        @pl.when(c == 0)
        def _():
            _edge_pipeline(lambda j: tabA_hbm.at[srcv.at[j]],
                           srcv, dstv, rows, gsems, ssems, acc, CHT)

        @pl.when(c == 1)
        def _():
            _edge_pipeline(lambda j: tabB_hbm.at[srcv.at[j]],
                           srcv, dstv, rows, gsems, ssems, acc, CHT)

        plsc.subcore_barrier()
        pltpu.sync_copy(acc.at[pl.ds(s * NPT, NPT)],
                        out_hbm.at[pl.ds(s * NPT, NPT), pl.ds(c * 64, 64)])

    return prop_kernel


@functools.lru_cache(maxsize=None)
def _make_prop_edge():
    @functools.partial(
        pl.kernel,
        out_type=jax.ShapeDtypeStruct((N_PAD, 128), jnp.float32),
        mesh=_mesh(),
        compiler_params=_sc_params(),
        scratch_types=_prop_scratch(CHW),
    )
    def prop_kernel(tab_hbm, src_hbm, dst_hbm, zeros_hbm, out_hbm,
                    srcv, dstv, r0, r1, r2, r3, r4, acc,
                    g0, g1, g2, g3, g4, s0, s1, s2, s3, s4):
        c = lax.axis_index("c")
        s = lax.axis_index("s")
        rows = [r0, r1, r2, r3, r4]
        gsems = [g0, g1, g2, g3, g4]
        ssems = [s0, s1, s2, s3, s4]

        pltpu.sync_copy(zeros_hbm.at[pl.ds(s * NPT, NPT)], acc.at[pl.ds(s * NPT, NPT)])
        pltpu.sync_copy(src_hbm.at[s, pl.ds(c * CHW, CHW)], srcv)
        pltpu.sync_copy(dst_hbm.at[s, pl.ds(c * CHW, CHW)], dstv)
        plsc.subcore_barrier()

        _edge_pipeline(lambda j: tab_hbm.at[srcv.at[j]],
                       srcv, dstv, rows, gsems, ssems, acc, CHW)

        plsc.subcore_barrier()
        pltpu.sync_copy(acc.at[pl.ds(s * NPT, NPT)],
                        out_hbm.at[pl.ds(s * NPT, NPT), pl.ds(c * 64, 64)])

    return prop_kernel


BLK = 1000


def _mlp_body(x_ref, wm_ref, bm_ref, degp_ref, htA_ref, htB_ref, dinv_ref):
    deg = degp_ref[:, 0:1] + degp_ref[:, 8:9] + 1.0
    dinv = lax.rsqrt(deg)
    h = jnp.dot(x_ref[...], wm_ref[...], preferred_element_type=jnp.float32)
    ht = jnp.maximum(h + bm_ref[...], 0.0) * dinv
    htA_ref[...] = ht[:, :64]
    htB_ref[...] = ht[:, 64:]
    dinv_ref[...] = jnp.broadcast_to(dinv, (dinv.shape[0], 8))


def _mid_body(acc_ref, htA_ref, htB_ref, dinv_ref, w1_ref, b1_ref, w2_ref, g_ref):
    dinv = dinv_ref[:, 0:1]
    ht = jnp.concatenate([htA_ref[...], htB_ref[...]], axis=1)
    u = (acc_ref[...] + ht) * dinv
    h1 = jnp.dot(u, w1_ref[...], preferred_element_type=jnp.float32)
    h1 = jnp.maximum(h1 + b1_ref[...], 0.0)
    g_ref[...] = jnp.dot(h1 * dinv, w2_ref[...], preferred_element_type=jnp.float32)


def _fin_body(acc_ref, g_ref, dinv_ref, b2_ref, out_ref):
    dinv = dinv_ref[:, 0:1]
    sg = acc_ref[:, :64] + acc_ref[:, 64:]
    out_ref[...] = (sg + g_ref[...]) * dinv + b2_ref[...]


def _tc_mlp(x, Wm, bm2, degp):
    return pl.pallas_call(
        _mlp_body,
        grid=(N // BLK,),
        in_specs=[
            pl.BlockSpec((BLK, 128), lambda i: (i, 0)),
            pl.BlockSpec((128, 128), lambda i: (0, 0)),
            pl.BlockSpec((1, 128), lambda i: (0, 0)),
            pl.BlockSpec((BLK, 128), lambda i: (i, 0)),
        ],
        out_specs=[
            pl.BlockSpec((BLK, 64), lambda i: (i, 0)),
            pl.BlockSpec((BLK, 64), lambda i: (i, 0)),
            pl.BlockSpec((BLK, 8), lambda i: (i, 0)),
        ],
        out_shape=[
            jax.ShapeDtypeStruct((N, 64), jnp.float32),
            jax.ShapeDtypeStruct((N, 64), jnp.float32),
            jax.ShapeDtypeStruct((N, 8), jnp.float32),
        ],
    )(x, Wm, bm2, degp)


def _tc_mid(acc1, htA, htB, dinv, W1, b12, W2):
    return pl.pallas_call(
        _mid_body,
        grid=(N // BLK,),
        in_specs=[
            pl.BlockSpec((BLK, 128), lambda i: (i, 0)),
            pl.BlockSpec((BLK, 64), lambda i: (i, 0)),
            pl.BlockSpec((BLK, 64), lambda i: (i, 0)),
            pl.BlockSpec((BLK, 8), lambda i: (i, 0)),
            pl.BlockSpec((128, 128), lambda i: (0, 0)),
            pl.BlockSpec((1, 128), lambda i: (0, 0)),
            pl.BlockSpec((128, 64), lambda i: (0, 0)),
        ],
        out_specs=pl.BlockSpec((BLK, 64), lambda i: (i, 0)),
        out_shape=jax.ShapeDtypeStruct((N, 64), jnp.float32),
    )(acc1, htA, htB, dinv, W1, b12, W2)


def _tc_fin(acc2, g, dinv, b22):
    return pl.pallas_call(
        _fin_body,
        grid=(N // BLK,),
        in_specs=[
            pl.BlockSpec((BLK, 128), lambda i: (i, 0)),
            pl.BlockSpec((BLK, 64), lambda i: (i, 0)),
            pl.BlockSpec((BLK, 8), lambda i: (i, 0)),
            pl.BlockSpec((1, 64), lambda i: (0, 0)),
        ],
        out_specs=pl.BlockSpec((BLK, 64), lambda i: (i, 0)),
        out_shape=jax.ShapeDtypeStruct((N, 64), jnp.float32),
    )(acc2, g, dinv, b22)


def kernel(x, edge_index, Wm, bm, W1, b1, W2, b2):
    src2 = edge_index[0].reshape(NS, CHT, K)
    dst2 = edge_index[1].reshape(NS, CHT, K)
    ones8 = jnp.ones((K, 8), jnp.float32)
    zeros8 = jnp.zeros((N_PAD, 8), jnp.float32)
    zeros64 = jnp.zeros((N_PAD, 64), jnp.float32)

    degp = _make_deg()(dst2, ones8, zeros8)
    htA, htB, dinv = _tc_mlp(x, Wm, bm.reshape(1, -1), degp)
    acc1 = _make_prop_col()(htA, htB, src2, dst2, zeros64)
    g = _tc_mid(acc1, htA, htB, dinv, W1, b1.reshape(1, -1), W2)
    acc2 = _make_prop_edge()(g, src2, dst2, zeros64)
    return _tc_fin(acc2, g, dinv, b2.reshape(1, -1))


# speedup vs baseline: 1.0859x; 1.0859x over previous
import functools

import jax
import jax.numpy as jnp
from jax import lax
from jax.experimental import pallas as pl
from jax.experimental.pallas import tpu as pltpu
from jax.experimental.pallas import tpu_sc as plsc

N = 10000
N_PAD = 10240
E = 320000
NC = 2
NS = 16
K = 80
CHT = E // (NS * K)
CHW = CHT // NC
NPT = N_PAD // NS
RB = 5

_mesh = lambda: plsc.VectorSubcoreMesh(core_axis_name="c", subcore_axis_name="s")
_sc_params = lambda: pltpu.CompilerParams(use_tc_tiling_on_sc=False)


@functools.lru_cache(maxsize=None)
def _make_deg():
    @functools.partial(
        pl.kernel,
        out_type=jax.ShapeDtypeStruct((N_PAD, 128), jnp.float32),
        mesh=_mesh(),
        compiler_params=_sc_params(),
        scratch_types=[
            pltpu.VMEM((CHW, K), jnp.int32),
            pltpu.VMEM((K, 8), jnp.float32),
            pltpu.VMEM_SHARED((N_PAD, 8), jnp.float32),
            pltpu.SemaphoreType.DMA,
        ],
    )
    def deg_kernel(dst_hbm, ones_hbm, zeros_hbm, out_hbm, dstv, ones_v, acc, ssem):
        c = lax.axis_index("c")
        s = lax.axis_index("s")

        pltpu.sync_copy(zeros_hbm.at[pl.ds(s * NPT, NPT)], acc.at[pl.ds(s * NPT, NPT)])
        pltpu.sync_copy(ones_hbm, ones_v)
        pltpu.sync_copy(dst_hbm.at[s, pl.ds(c * CHW, CHW)], dstv)
        plsc.subcore_barrier()

        def group(g, _):
            for t in range(RB):
                pltpu.async_copy(ones_v, acc.at[dstv.at[g * RB + t]], ssem,
                                 add=True)
            for t in range(RB):
                pltpu.make_async_copy(ones_v, acc.at[dstv.at[0]], ssem).wait()
            return 0

        lax.fori_loop(0, CHW // RB, group, 0)
        plsc.subcore_barrier()
        pltpu.sync_copy(acc.at[pl.ds(s * NPT, NPT)],
                        out_hbm.at[pl.ds(s * NPT, NPT), pl.ds(c * 8, 8)])

    return deg_kernel


def _edge_pipeline(mk_src, srcv, dstv, rows, gsems, ssems, acc, n_chunks):
    M = n_chunks // RB

    def gissue(j, b):
        pltpu.async_copy(mk_src(j), rows[b], gsems[b])

    def gwait(b):
        pltpu.make_async_copy(mk_src(0), rows[b], gsems[b]).wait()

    def sissue(j, b):
        pltpu.async_copy(rows[b], acc.at[dstv.at[j]], ssems[b], add=True)

    def swait(b):
        pltpu.make_async_copy(rows[b], acc.at[dstv.at[0]], ssems[b]).wait()

    for b in range(3):
        gissue(b, b)
    for b in range(RB):
        b2 = (b - 2) % RB
        if b >= 2:
            swait(b2)
        gissue(b + 3, b2)
        gwait(b)
        sissue(b, b)

    def step(m, _):
        for b in range(RB):
            j = m * RB + b
            b2 = (b - 2) % RB
            swait(b2)
            gissue(j + 3, b2)
            gwait(b)
            sissue(j, b)
        return 0

    lax.fori_loop(1, M - 1, step, 0)

    base = (M - 1) * RB
    for b in range(RB):
        if b < 2:
            b2 = (b - 2) % RB
            swait(b2)
            gissue(base + b + 3, b2)
        gwait(b)
        sissue(base + b, b)
    for b in range(RB):
        swait(b)


def _prop_scratch(n_chunks):
    return [
        pltpu.VMEM((n_chunks, K), jnp.int32),
        pltpu.VMEM((n_chunks, K), jnp.int32),
    ] + [pltpu.VMEM((K, 64), jnp.float32) for _ in range(RB)] + [
        pltpu.VMEM_SHARED((N_PAD, 64), jnp.float32),
    ] + [pltpu.SemaphoreType.DMA for _ in range(2 * RB)]


@functools.lru_cache(maxsize=None)
def _make_prop_col():
    @functools.partial(
        pl.kernel,
        out_type=jax.ShapeDtypeStruct((N_PAD, 128), jnp.float32),
        mesh=_mesh(),
        compiler_params=_sc_params(),
        scratch_types=_prop_scratch(CHT),
    )
    def prop_kernel(tabA_hbm, tabB_hbm, src_hbm, dst_hbm, zeros_hbm, out_hbm,
                    srcv, dstv, r0, r1, r2, r3, r4, acc,
                    g0, g1, g2, g3, g4, s0, s1, s2, s3, s4):
        c = lax.axis_index("c")
        s = lax.axis_index("s")
        rows = [r0, r1, r2, r3, r4]
        gsems = [g0, g1, g2, g3, g4]
        ssems = [s0, s1, s2, s3, s4]

        pltpu.sync_copy(zeros_hbm.at[pl.ds(s * NPT, NPT)], acc.at[pl.ds(s * NPT, NPT)])
        pltpu.sync_copy(src_hbm.at[s], srcv)
        pltpu.sync_copy(dst_hbm.at[s], dstv)
        plsc.subcore_barrier()

        @pl.when(c == 0)
        def _():
            _edge_pipeline(lambda j: tabA_hbm.at[srcv.at[j]],
                           srcv, dstv, rows, gsems, ssems, acc, CHT)

        @pl.when(c == 1)
        def _():
            _edge_pipeline(lambda j: tabB_hbm.at[srcv.at[j]],
                           srcv, dstv, rows, gsems, ssems, acc, CHT)

        plsc.subcore_barrier()
        pltpu.sync_copy(acc.at[pl.ds(s * NPT, NPT)],
                        out_hbm.at[pl.ds(s * NPT, NPT), pl.ds(c * 64, 64)])

    return prop_kernel


@functools.lru_cache(maxsize=None)
def _make_prop_edge():
    @functools.partial(
        pl.kernel,
        out_type=jax.ShapeDtypeStruct((N_PAD, 128), jnp.float32),
        mesh=_mesh(),
        compiler_params=_sc_params(),
        scratch_types=_prop_scratch(CHW),
    )
    def prop_kernel(tab_hbm, src_hbm, dst_hbm, zeros_hbm, out_hbm,
                    srcv, dstv, r0, r1, r2, r3, r4, acc,
                    g0, g1, g2, g3, g4, s0, s1, s2, s3, s4):
        c = lax.axis_index("c")
        s = lax.axis_index("s")
        rows = [r0, r1, r2, r3, r4]
        gsems = [g0, g1, g2, g3, g4]
        ssems = [s0, s1, s2, s3, s4]

        pltpu.sync_copy(zeros_hbm.at[pl.ds(s * NPT, NPT)], acc.at[pl.ds(s * NPT, NPT)])
        pltpu.sync_copy(src_hbm.at[s, pl.ds(c * CHW, CHW)], srcv)
        pltpu.sync_copy(dst_hbm.at[s, pl.ds(c * CHW, CHW)], dstv)
        plsc.subcore_barrier()

        _edge_pipeline(lambda j: tab_hbm.at[srcv.at[j]],
                       srcv, dstv, rows, gsems, ssems, acc, CHW)

        plsc.subcore_barrier()
        pltpu.sync_copy(acc.at[pl.ds(s * NPT, NPT)],
                        out_hbm.at[pl.ds(s * NPT, NPT), pl.ds(c * 64, 64)])

    return prop_kernel


BLK = 2000


def _mlp_body(x_ref, wm_ref, bm_ref, degp_ref, htA_ref, htB_ref, dinv_ref):
    deg = degp_ref[:, 0:1] + degp_ref[:, 8:9] + 1.0
    dinv = lax.rsqrt(deg)
    h = jnp.dot(x_ref[...], wm_ref[...], preferred_element_type=jnp.float32)
    ht = jnp.maximum(h + bm_ref[...], 0.0) * dinv
    htA_ref[...] = ht[:, :64]
    htB_ref[...] = ht[:, 64:]
    dinv_ref[...] = jnp.broadcast_to(dinv, (dinv.shape[0], 8))


def _mid_body(acc_ref, htA_ref, htB_ref, dinv_ref, w1_ref, b1_ref, w2_ref, g_ref):
    dinv = dinv_ref[:, 0:1]
    ht = jnp.concatenate([htA_ref[...], htB_ref[...]], axis=1)
    u = (acc_ref[...] + ht) * dinv
    h1 = jnp.dot(u, w1_ref[...], preferred_element_type=jnp.float32)
    h1 = jnp.maximum(h1 + b1_ref[...], 0.0)
    g_ref[...] = jnp.dot(h1 * dinv, w2_ref[...], preferred_element_type=jnp.float32)


def _fin_body(acc_ref, g_ref, dinv_ref, b2_ref, out_ref):
    dinv = dinv_ref[:, 0:1]
    sg = acc_ref[:, :64] + acc_ref[:, 64:]
    out_ref[...] = (sg + g_ref[...]) * dinv + b2_ref[...]


def _tc_mlp(x, Wm, bm2, degp):
    return pl.pallas_call(
        _mlp_body,
        grid=(N // BLK,),
        in_specs=[
            pl.BlockSpec((BLK, 128), lambda i: (i, 0)),
            pl.BlockSpec((128, 128), lambda i: (0, 0)),
            pl.BlockSpec((1, 128), lambda i: (0, 0)),
            pl.BlockSpec((BLK, 128), lambda i: (i, 0)),
        ],
        out_specs=[
            pl.BlockSpec((BLK, 64), lambda i: (i, 0)),
            pl.BlockSpec((BLK, 64), lambda i: (i, 0)),
            pl.BlockSpec((BLK, 8), lambda i: (i, 0)),
        ],
        out_shape=[
            jax.ShapeDtypeStruct((N, 64), jnp.float32),
            jax.ShapeDtypeStruct((N, 64), jnp.float32),
            jax.ShapeDtypeStruct((N, 8), jnp.float32),
        ],
    )(x, Wm, bm2, degp)


def _tc_mid(acc1, htA, htB, dinv, W1, b12, W2):
    return pl.pallas_call(
        _mid_body,
        grid=(N // BLK,),
        in_specs=[
            pl.BlockSpec((BLK, 128), lambda i: (i, 0)),
            pl.BlockSpec((BLK, 64), lambda i: (i, 0)),
            pl.BlockSpec((BLK, 64), lambda i: (i, 0)),
            pl.BlockSpec((BLK, 8), lambda i: (i, 0)),
            pl.BlockSpec((128, 128), lambda i: (0, 0)),
            pl.BlockSpec((1, 128), lambda i: (0, 0)),
            pl.BlockSpec((128, 64), lambda i: (0, 0)),
        ],
        out_specs=pl.BlockSpec((BLK, 64), lambda i: (i, 0)),
        out_shape=jax.ShapeDtypeStruct((N, 64), jnp.float32),
    )(acc1, htA, htB, dinv, W1, b12, W2)


def _detile_body(ei_ref, src_ref, dst_ref):
    src_ref[...] = ei_ref[0, :]
    dst_ref[...] = ei_ref[1, :]


def _tc_detile(edge_index):
    return pl.pallas_call(
        _detile_body,
        out_shape=[
            jax.ShapeDtypeStruct((E,), jnp.int32),
            jax.ShapeDtypeStruct((E,), jnp.int32),
        ],
    )(edge_index)


def _tc_fin(acc2, g, dinv, b22):
    return pl.pallas_call(
        _fin_body,
        grid=(N // BLK,),
        in_specs=[
            pl.BlockSpec((BLK, 128), lambda i: (i, 0)),
            pl.BlockSpec((BLK, 64), lambda i: (i, 0)),
            pl.BlockSpec((BLK, 8), lambda i: (i, 0)),
            pl.BlockSpec((1, 64), lambda i: (0, 0)),
        ],
        out_specs=pl.BlockSpec((BLK, 64), lambda i: (i, 0)),
        out_shape=jax.ShapeDtypeStruct((N, 64), jnp.float32),
    )(acc2, g, dinv, b22)


def kernel(x, edge_index, Wm, bm, W1, b1, W2, b2):
    src_flat, dst_flat = _tc_detile(edge_index)
    src2 = src_flat.reshape(NS, CHT, K)
    dst2 = dst_flat.reshape(NS, CHT, K)
    ones8 = jnp.ones((K, 8), jnp.float32)
    zeros8 = jnp.zeros((N_PAD, 8), jnp.float32)
    zeros64 = jnp.zeros((N_PAD, 64), jnp.float32)

    degp = _make_deg()(dst2, ones8, zeros8)
    htA, htB, dinv = _tc_mlp(x, Wm, bm.reshape(1, -1), degp)
    acc1 = _make_prop_col()(htA, htB, src2, dst2, zeros64)
    g = _tc_mid(acc1, htA, htB, dinv, W1, b1.reshape(1, -1), W2)
    acc2 = _make_prop_edge()(g, src2, dst2, zeros64)
    return _tc_fin(acc2, g, dinv, b2.reshape(1, -1))


# speedup vs baseline: 1.0907x; 1.0045x over previous
import functools

import jax
import jax.numpy as jnp
from jax import lax
from jax.experimental import pallas as pl
from jax.experimental.pallas import tpu as pltpu
from jax.experimental.pallas import tpu_sc as plsc

N = 10000
N_PAD = 10240
E = 320000
NC = 2
NS = 16
K = 80
CHT = E // (NS * K)
CHW = CHT // NC
NPT = N_PAD // NS
RB = 5

_mesh = lambda: plsc.VectorSubcoreMesh(core_axis_name="c", subcore_axis_name="s")
_sc_params = lambda: pltpu.CompilerParams(use_tc_tiling_on_sc=False)


@functools.lru_cache(maxsize=None)
def _make_deg():
    @functools.partial(
        pl.kernel,
        out_type=jax.ShapeDtypeStruct((N_PAD, 128), jnp.float32),
        mesh=_mesh(),
        compiler_params=_sc_params(),
        scratch_types=[
            pltpu.VMEM((CHW, K), jnp.int32),
            pltpu.VMEM((K, 8), jnp.float32),
            pltpu.VMEM_SHARED((N_PAD, 8), jnp.float32),
            pltpu.SemaphoreType.DMA,
        ],
    )
    def deg_kernel(dst_hbm, ones_hbm, zeros_hbm, out_hbm, dstv, ones_v, acc, ssem):
        c = lax.axis_index("c")
        s = lax.axis_index("s")

        pltpu.sync_copy(zeros_hbm.at[pl.ds(s * NPT, NPT)], acc.at[pl.ds(s * NPT, NPT)])
        pltpu.sync_copy(ones_hbm, ones_v)
        pltpu.sync_copy(dst_hbm.at[s, pl.ds(c * CHW, CHW)], dstv)
        plsc.subcore_barrier()

        GD = 25

        def group(g, _):
            for t in range(GD):
                pltpu.async_copy(ones_v, acc.at[dstv.at[g * GD + t]], ssem,
                                 add=True)
            for t in range(GD):
                pltpu.make_async_copy(ones_v, acc.at[dstv.at[0]], ssem).wait()
            return 0

        lax.fori_loop(0, CHW // GD, group, 0)
        plsc.subcore_barrier()
        pltpu.sync_copy(acc.at[pl.ds(s * NPT, NPT)],
                        out_hbm.at[pl.ds(s * NPT, NPT), pl.ds(c * 8, 8)])

    return deg_kernel


def _edge_pipeline(mk_src, srcv, dstv, rows, gsems, ssems, acc, n_chunks):
    M = n_chunks // RB

    def gissue(j, b):
        pltpu.async_copy(mk_src(j), rows[b], gsems[b])

    def gwait(b):
        pltpu.make_async_copy(mk_src(0), rows[b], gsems[b]).wait()

    def sissue(j, b):
        pltpu.async_copy(rows[b], acc.at[dstv.at[j]], ssems[b], add=True)

    def swait(b):
        pltpu.make_async_copy(rows[b], acc.at[dstv.at[0]], ssems[b]).wait()

    for b in range(3):
        gissue(b, b)
    for b in range(RB):
        b2 = (b - 2) % RB
        if b >= 2:
            swait(b2)
        gissue(b + 3, b2)
        gwait(b)
        sissue(b, b)

    def step(m, _):
        for b in range(RB):
            j = m * RB + b
            b2 = (b - 2) % RB
            swait(b2)
            gissue(j + 3, b2)
            gwait(b)
            sissue(j, b)
        return 0

    lax.fori_loop(1, M - 1, step, 0)

    base = (M - 1) * RB
    for b in range(RB):
        if b < 2:
            b2 = (b - 2) % RB
            swait(b2)
            gissue(base + b + 3, b2)
        gwait(b)
        sissue(base + b, b)
    for b in range(RB):
        swait(b)


def _prop_scratch(n_chunks):
    return [
        pltpu.VMEM((n_chunks, K), jnp.int32),
        pltpu.VMEM((n_chunks, K), jnp.int32),
    ] + [pltpu.VMEM((K, 64), jnp.float32) for _ in range(RB)] + [
        pltpu.VMEM_SHARED((N_PAD, 64), jnp.float32),
    ] + [pltpu.SemaphoreType.DMA for _ in range(2 * RB)]


@functools.lru_cache(maxsize=None)
def _make_prop_col():
    @functools.partial(
        pl.kernel,
        out_type=jax.ShapeDtypeStruct((N_PAD, 128), jnp.float32),
        mesh=_mesh(),
        compiler_params=_sc_params(),
        scratch_types=_prop_scratch(CHT),
    )
    def prop_kernel(tabA_hbm, tabB_hbm, src_hbm, dst_hbm, zeros_hbm, out_hbm,
                    srcv, dstv, r0, r1, r2, r3, r4, acc,
                    g0, g1, g2, g3, g4, s0, s1, s2, s3, s4):
        c = lax.axis_index("c")
        s = lax.axis_index("s")
        rows = [r0, r1, r2, r3, r4]
        gsems = [g0, g1, g2, g3, g4]
        ssems = [s0, s1, s2, s3, s4]

        pltpu.sync_copy(zeros_hbm.at[pl.ds(s * NPT, NPT)], acc.at[pl.ds(s * NPT, NPT)])
        pltpu.sync_copy(src_hbm.at[s], srcv)
        pltpu.sync_copy(dst_hbm.at[s], dstv)
        plsc.subcore_barrier()

        @pl.when(c == 0)
        def _():
            _edge_pipeline(lambda j: tabA_hbm.at[srcv.at[j]],
                           srcv, dstv, rows, gsems, ssems, acc, CHT)

        @pl.when(c == 1)
        def _():
            _edge_pipeline(lambda j: tabB_hbm.at[srcv.at[j]],
                           srcv, dstv, rows, gsems, ssems, acc, CHT)

        plsc.subcore_barrier()
        pltpu.sync_copy(acc.at[pl.ds(s * NPT, NPT)],
                        out_hbm.at[pl.ds(s * NPT, NPT), pl.ds(c * 64, 64)])

    return prop_kernel


@functools.lru_cache(maxsize=None)
def _make_prop_edge():
    @functools.partial(
        pl.kernel,
        out_type=jax.ShapeDtypeStruct((N_PAD, 128), jnp.float32),
        mesh=_mesh(),
        compiler_params=_sc_params(),
        scratch_types=_prop_scratch(CHW),
    )
    def prop_kernel(tab_hbm, src_hbm, dst_hbm, zeros_hbm, out_hbm,
                    srcv, dstv, r0, r1, r2, r3, r4, acc,
                    g0, g1, g2, g3, g4, s0, s1, s2, s3, s4):
        c = lax.axis_index("c")
        s = lax.axis_index("s")
        rows = [r0, r1, r2, r3, r4]
        gsems = [g0, g1, g2, g3, g4]
        ssems = [s0, s1, s2, s3, s4]

        pltpu.sync_copy(zeros_hbm.at[pl.ds(s * NPT, NPT)], acc.at[pl.ds(s * NPT, NPT)])
        pltpu.sync_copy(src_hbm.at[s, pl.ds(c * CHW, CHW)], srcv)
        pltpu.sync_copy(dst_hbm.at[s, pl.ds(c * CHW, CHW)], dstv)
        plsc.subcore_barrier()

        _edge_pipeline(lambda j: tab_hbm.at[srcv.at[j]],
                       srcv, dstv, rows, gsems, ssems, acc, CHW)

        plsc.subcore_barrier()
        pltpu.sync_copy(acc.at[pl.ds(s * NPT, NPT)],
                        out_hbm.at[pl.ds(s * NPT, NPT), pl.ds(c * 64, 64)])

    return prop_kernel


BLK = 2000


def _mlp_body(x_ref, wm_ref, bm_ref, degp_ref, htA_ref, htB_ref, dinv_ref):
    deg = degp_ref[:, 0:1] + degp_ref[:, 8:9] + 1.0
    dinv = lax.rsqrt(deg)
    h = jnp.dot(x_ref[...], wm_ref[...], preferred_element_type=jnp.float32)
    ht = jnp.maximum(h + bm_ref[...], 0.0) * dinv
    htA_ref[...] = ht[:, :64]
    htB_ref[...] = ht[:, 64:]
    dinv_ref[...] = jnp.broadcast_to(dinv, (dinv.shape[0], 8))


def _mid_body(acc_ref, htA_ref, htB_ref, dinv_ref, w1_ref, b1_ref, w2_ref, g_ref):
    dinv = dinv_ref[:, 0:1]
    ht = jnp.concatenate([htA_ref[...], htB_ref[...]], axis=1)
    u = (acc_ref[...] + ht) * dinv
    h1 = jnp.dot(u, w1_ref[...], preferred_element_type=jnp.float32)
    h1 = jnp.maximum(h1 + b1_ref[...], 0.0)
    g_ref[...] = jnp.dot(h1 * dinv, w2_ref[...], preferred_element_type=jnp.float32)


def _fin_body(acc_ref, g_ref, dinv_ref, b2_ref, out_ref):
    dinv = dinv_ref[:, 0:1]
    sg = acc_ref[:, :64] + acc_ref[:, 64:]
    out_ref[...] = (sg + g_ref[...]) * dinv + b2_ref[...]


def _tc_mlp(x, Wm, bm2, degp):
    return pl.pallas_call(
        _mlp_body,
        grid=(N // BLK,),
        in_specs=[
            pl.BlockSpec((BLK, 128), lambda i: (i, 0)),
            pl.BlockSpec((128, 128), lambda i: (0, 0)),
            pl.BlockSpec((1, 128), lambda i: (0, 0)),
            pl.BlockSpec((BLK, 128), lambda i: (i, 0)),
        ],
        out_specs=[
            pl.BlockSpec((BLK, 64), lambda i: (i, 0)),
            pl.BlockSpec((BLK, 64), lambda i: (i, 0)),
            pl.BlockSpec((BLK, 8), lambda i: (i, 0)),
        ],
        out_shape=[
            jax.ShapeDtypeStruct((N, 64), jnp.float32),
            jax.ShapeDtypeStruct((N, 64), jnp.float32),
            jax.ShapeDtypeStruct((N, 8), jnp.float32),
        ],
    )(x, Wm, bm2, degp)


def _tc_mid(acc1, htA, htB, dinv, W1, b12, W2):
    return pl.pallas_call(
        _mid_body,
        grid=(N // BLK,),
        in_specs=[
            pl.BlockSpec((BLK, 128), lambda i: (i, 0)),
            pl.BlockSpec((BLK, 64), lambda i: (i, 0)),
            pl.BlockSpec((BLK, 64), lambda i: (i, 0)),
            pl.BlockSpec((BLK, 8), lambda i: (i, 0)),
            pl.BlockSpec((128, 128), lambda i: (0, 0)),
            pl.BlockSpec((1, 128), lambda i: (0, 0)),
            pl.BlockSpec((128, 64), lambda i: (0, 0)),
        ],
        out_specs=pl.BlockSpec((BLK, 64), lambda i: (i, 0)),
        out_shape=jax.ShapeDtypeStruct((N, 64), jnp.float32),
    )(acc1, htA, htB, dinv, W1, b12, W2)


def _detile_body(ei_ref, src_ref, dst_ref):
    src_ref[...] = ei_ref[0, :]
    dst_ref[...] = ei_ref[1, :]


def _tc_detile(edge_index):
    return pl.pallas_call(
        _detile_body,
        out_shape=[
            jax.ShapeDtypeStruct((E,), jnp.int32),
            jax.ShapeDtypeStruct((E,), jnp.int32),
        ],
    )(edge_index)


def _tc_fin(acc2, g, dinv, b22):
    return pl.pallas_call(
        _fin_body,
        grid=(N // BLK,),
        in_specs=[
            pl.BlockSpec((BLK, 128), lambda i: (i, 0)),
            pl.BlockSpec((BLK, 64), lambda i: (i, 0)),
            pl.BlockSpec((BLK, 8), lambda i: (i, 0)),
            pl.BlockSpec((1, 64), lambda i: (0, 0)),
        ],
        out_specs=pl.BlockSpec((BLK, 64), lambda i: (i, 0)),
        out_shape=jax.ShapeDtypeStruct((N, 64), jnp.float32),
    )(acc2, g, dinv, b22)


def kernel(x, edge_index, Wm, bm, W1, b1, W2, b2):
    src_flat, dst_flat = _tc_detile(edge_index)
    src2 = src_flat.reshape(NS, CHT, K)
    dst2 = dst_flat.reshape(NS, CHT, K)
    ones8 = jnp.ones((K, 8), jnp.float32)
    zeros8 = jnp.zeros((N_PAD, 8), jnp.float32)
    zeros64 = jnp.zeros((N_PAD, 64), jnp.float32)

    degp = _make_deg()(dst2, ones8, zeros8)
    htA, htB, dinv = _tc_mlp(x, Wm, bm.reshape(1, -1), degp)
    acc1 = _make_prop_col()(htA, htB, src2, dst2, zeros64)
    g = _tc_mid(acc1, htA, htB, dinv, W1, b1.reshape(1, -1), W2)
    acc2 = _make_prop_edge()(g, src2, dst2, zeros64)
    return _tc_fin(acc2, g, dinv, b2.reshape(1, -1))
